# Initial kernel scaffold; baseline (speedup 1.0000x reference)
import jax
import jax.numpy as jnp
from jax.experimental import pallas as pl


def kernel(x, edge_index, edge_weight, W1, b1, W2, b2):
    raise NotImplementedError("write your pallas kernel here")


# speedup vs baseline: 3.1980x; 3.1980x over previous
import functools

import jax
import jax.numpy as jnp
from jax import lax
from jax.experimental import pallas as pl
from jax.experimental.pallas import tpu as pltpu
from jax.experimental.pallas import tpu_sc as plsc

NUM_CORES = 2
NUM_SUBCORES = 16
LANES = 16
NUM_WORKERS = NUM_CORES * NUM_SUBCORES

FEAT = 128
ACC_W = 144
K_EDGES = 80
ZROWS = 125


def _sc_spmm(x, src, dst, ew, n_nodes, n_edges):
    e_per_w = n_edges // NUM_WORKERS
    n_chunks = e_per_w // K_EDGES
    rows_per_sub = n_nodes // NUM_SUBCORES
    nz = rows_per_sub // ZROWS

    mesh = plsc.VectorSubcoreMesh(core_axis_name="c", subcore_axis_name="s")

    @functools.partial(
        pl.kernel,
        out_type=jax.ShapeDtypeStruct((NUM_CORES, n_nodes, ACC_W), jnp.float32),
        mesh=mesh,
        scratch_types=[
            pltpu.VMEM((K_EDGES,), jnp.int32),
            pltpu.VMEM((K_EDGES,), jnp.int32),
            pltpu.VMEM((K_EDGES,), jnp.float32),
            pltpu.VMEM((K_EDGES, FEAT), jnp.float32),
            pltpu.VMEM((ZROWS, ACC_W), jnp.float32),
            pltpu.VMEM_SHARED((n_nodes, ACC_W), jnp.float32),
            pltpu.SemaphoreType.DMA,
        ],
        compiler_params=pltpu.CompilerParams(
            use_tc_tiling_on_sc=False, needs_layout_passes=False),
    )
    def spmm(x_hbm, src_hbm, dst_hbm, ew_hbm, out_hbm,
             src_v, dst_v, ew_v, gbuf, sbuf, acc, sem):
        c = lax.axis_index("c")
        s = lax.axis_index("s")
        wid = c * NUM_SUBCORES + s

        zero16 = jnp.zeros((LANES,), jnp.float32)
        lane0 = lax.iota(jnp.int32, LANES) == 0

        def zrow(r, _):
            for j in range(ACC_W // LANES):
                sbuf[r, pl.ds(j * LANES, LANES)] = zero16
            return 0
        lax.fori_loop(0, ZROWS, zrow, 0)
        for z in range(nz):
            pltpu.sync_copy(
                sbuf,
                acc.at[pl.ds(s * rows_per_sub + z * ZROWS, ZROWS)])
        plsc.subcore_barrier()

        def chunk(g, _):
            base = wid * e_per_w + g * K_EDGES
            pltpu.sync_copy(src_hbm.at[pl.ds(base, K_EDGES)], src_v)
            pltpu.sync_copy(dst_hbm.at[pl.ds(base, K_EDGES)], dst_v)
            pltpu.sync_copy(ew_hbm.at[pl.ds(base, K_EDGES)], ew_v)
            pltpu.async_copy(x_hbm.at[src_v], gbuf, sem).wait()

            def edge(e, _):
                wv = plsc.load_gather(ew_v, [jnp.full((LANES,), e, jnp.int32)])
                for j in range(FEAT // LANES):
                    sbuf[e, pl.ds(j * LANES, LANES)] = (
                        gbuf[e, pl.ds(j * LANES, LANES)] * wv)
                sbuf[e, pl.ds(FEAT, LANES)] = jnp.where(lane0, wv, zero16)
                return 0
            lax.fori_loop(0, K_EDGES, edge, 0)

            pltpu.sync_copy(sbuf.at[pl.ds(0, K_EDGES)], acc.at[dst_v], add=True)
            return 0
        lax.fori_loop(0, n_chunks, chunk, 0)
        plsc.subcore_barrier()

        pltpu.sync_copy(
            acc.at[pl.ds(s * rows_per_sub, rows_per_sub)],
            out_hbm.at[c, pl.ds(s * rows_per_sub, rows_per_sub)])

    return spmm(x, src, dst, ew)


def _tc_finish(partials, w1, b1, w2, b2):
    n_nodes = partials.shape[1]
    nout = w2.shape[1]

    def body(p_ref, w1_ref, b1_ref, w2_ref, b2_ref, o_ref):
        s = p_ref[0] + p_ref[1]
        w12 = jnp.dot(w1_ref[...], w2_ref[...],
                      preferred_element_type=jnp.float32)
        v = jnp.dot(b1_ref[...], w2_ref[...],
                    preferred_element_type=jnp.float32)
        o_ref[...] = (jnp.dot(s[:, :FEAT], w12,
                              preferred_element_type=jnp.float32)
                      + s[:, FEAT:FEAT + 1] * v + b2_ref[...])

    return pl.pallas_call(
        body,
        out_shape=jax.ShapeDtypeStruct((n_nodes, nout), jnp.float32),
    )(partials, w1, b1.reshape(1, -1), w2, b2.reshape(1, -1))


def kernel(x, edge_index, edge_weight, W1, b1, W2, b2):
    n_nodes = x.shape[0]
    n_edges = edge_index.shape[1]
    src = edge_index[1].astype(jnp.int32)
    dst = edge_index[0].astype(jnp.int32)
    partials = _sc_spmm(x, src, dst, edge_weight.astype(jnp.float32),
                        n_nodes, n_edges)
    return _tc_finish(partials, W1, b1, W2, b2)


# speedup vs baseline: 4.1312x; 1.2918x over previous
import functools

import jax
import jax.numpy as jnp
from jax import lax
from jax.experimental import pallas as pl
from jax.experimental.pallas import tpu as pltpu
from jax.experimental.pallas import tpu_sc as plsc

NUM_CORES = 2
NUM_SUBCORES = 16
LANES = 16
NUM_WORKERS = NUM_CORES * NUM_SUBCORES

FEAT = 128
ACC_W = 144
K_EDGES = 64
E_PER_W = 10240


def _sc_spmm(x, idx3, n_nodes):
    n_chunks = idx3.shape[1]
    n_pairs = n_chunks // 2
    rows_per_sub = n_nodes // NUM_SUBCORES

    mesh = plsc.VectorSubcoreMesh(core_axis_name="c", subcore_axis_name="s")

    @functools.partial(
        pl.kernel,
        out_type=jax.ShapeDtypeStruct((NUM_CORES, n_nodes, ACC_W), jnp.float32),
        mesh=mesh,
        scratch_types=[
            pltpu.VMEM((2, 3, K_EDGES), jnp.int32),
            pltpu.VMEM((2, K_EDGES), jnp.int32),
            pltpu.VMEM((2, K_EDGES, FEAT), jnp.float32),
            pltpu.VMEM((2, K_EDGES, ACC_W), jnp.float32),
            pltpu.VMEM_SHARED((n_nodes, ACC_W), jnp.float32),
            pltpu.SemaphoreType.DMA,
            pltpu.SemaphoreType.DMA,
            pltpu.SemaphoreType.DMA,
            pltpu.SemaphoreType.DMA,
        ],
        compiler_params=pltpu.CompilerParams(
            use_tc_tiling_on_sc=False, needs_layout_passes=False),
    )
    def spmm(x_hbm, idx3_hbm, out_hbm,
             idx_v, dstbuf, gbuf, sbuf, acc,
             sem_g0, sem_g1, sem_s0, sem_s1):
        c = lax.axis_index("c")
        s = lax.axis_index("s")
        wid = c * NUM_SUBCORES + s
        sem_g = (sem_g0, sem_g1)
        sem_s = (sem_s0, sem_s1)

        zero16 = jnp.zeros((LANES,), jnp.float32)
        lane0 = lax.iota(jnp.int32, LANES) == 0

        def zrow(r, _):
            for j in range(ACC_W // LANES):
                sbuf[0, r, pl.ds(j * LANES, LANES)] = zero16
            return 0
        lax.fori_loop(0, K_EDGES, zrow, 0)
        done = 0
        while done < rows_per_sub:
            step = min(K_EDGES, rows_per_sub - done)
            pltpu.sync_copy(
                sbuf.at[0, pl.ds(0, step)],
                acc.at[pl.ds(s * rows_per_sub + done, step)])
            done += step
        plsc.subcore_barrier()

        def load_idx(g, b):
            pltpu.sync_copy(idx3_hbm.at[wid, g], idx_v.at[b])

        def start_gather(b):
            pltpu.async_copy(x_hbm.at[idx_v.at[b, 0]], gbuf.at[b], sem_g[b])

        def wait_gather(b):
            pltpu.make_async_copy(x_hbm.at[idx_v.at[b, 0]], gbuf.at[b],
                                  sem_g[b]).wait()

        def start_scatter(b):
            pltpu.async_copy(sbuf.at[b], acc.at[dstbuf.at[b]], sem_s[b],
                             add=True)

        def wait_scatter(b):
            pltpu.make_async_copy(sbuf.at[b], acc.at[dstbuf.at[b]],
                                  sem_s[b]).wait()

        def scale(b):
            ew_row = idx_v.at[b, 2]

            def edge(e, _):
                wbits = plsc.load_gather(
                    ew_row, [jnp.full((LANES,), e, jnp.int32)])
                wv = plsc.bitcast(wbits, jnp.float32)
                for j in range(FEAT // LANES):
                    sbuf[b, e, pl.ds(j * LANES, LANES)] = (
                        gbuf[b, e, pl.ds(j * LANES, LANES)] * wv)
                sbuf[b, e, pl.ds(FEAT, LANES)] = jnp.where(lane0, wv, zero16)
                return 0
            lax.fori_loop(0, K_EDGES, edge, 0)

        def save_dst(b):
            for q in range(K_EDGES // LANES):
                dstbuf[b, pl.ds(q * LANES, LANES)] = (
                    idx_v[b, 1, pl.ds(q * LANES, LANES)])

        load_idx(0, 0)
        load_idx(1, 1)
        start_gather(0)
        start_gather(1)

        def pair(t, _):
            for b in (0, 1):
                g = 2 * t + b
                wait_gather(b)

                @pl.when(t > 0)
                def _():
                    wait_scatter(b)

                scale(b)
                save_dst(b)

                @pl.when(g + 2 < n_chunks)
                def _():
                    load_idx(g + 2, b)
                    start_gather(b)

                start_scatter(b)
            return 0
        lax.fori_loop(0, n_pairs, pair, 0)
        wait_scatter(0)
        wait_scatter(1)
        plsc.subcore_barrier()

        pltpu.sync_copy(
            acc.at[pl.ds(s * rows_per_sub, rows_per_sub)],
            out_hbm.at[c, pl.ds(s * rows_per_sub, rows_per_sub)])

    return spmm(x, idx3)


def _tc_finish(partials, w1, b1, w2, b2):
    n_nodes = partials.shape[1]
    nout = w2.shape[1]

    def body(p_ref, w1_ref, b1_ref, w2_ref, b2_ref, o_ref):
        s = p_ref[0] + p_ref[1]
        w12 = jnp.dot(w1_ref[...], w2_ref[...],
                      preferred_element_type=jnp.float32)
        v = jnp.dot(b1_ref[...], w2_ref[...],
                    preferred_element_type=jnp.float32)
        o_ref[...] = (jnp.dot(s[:, :FEAT], w12,
                              preferred_element_type=jnp.float32)
                      + s[:, FEAT:FEAT + 1] * v + b2_ref[...])

    return pl.pallas_call(
        body,
        out_shape=jax.ShapeDtypeStruct((n_nodes, nout), jnp.float32),
    )(partials, w1, b1.reshape(1, -1), w2, b2.reshape(1, -1))


def kernel(x, edge_index, edge_weight, W1, b1, W2, b2):
    n_nodes = x.shape[0]
    n_edges = edge_index.shape[1]
    e_pad = NUM_WORKERS * E_PER_W - n_edges
    blk = (NUM_WORKERS, E_PER_W // K_EDGES, K_EDGES)
    pad = lambda a: jnp.concatenate([a, jnp.zeros((e_pad,), a.dtype)])
    src = pad(edge_index[1].astype(jnp.int32)).reshape(blk)
    dst = pad(edge_index[0].astype(jnp.int32)).reshape(blk)
    ewb = lax.bitcast_convert_type(
        pad(edge_weight.astype(jnp.float32)), jnp.int32).reshape(blk)
    idx3 = jnp.stack([src, dst, ewb], axis=2)
    partials = _sc_spmm(x, idx3, n_nodes)
    return _tc_finish(partials, W1, b1, W2, b2)


# speedup vs baseline: 4.6021x; 1.1140x over previous
import functools

import jax
import jax.numpy as jnp
from jax import lax
from jax.experimental import pallas as pl
from jax.experimental.pallas import tpu as pltpu
from jax.experimental.pallas import tpu_sc as plsc

NUM_CORES = 2
NUM_SUBCORES = 16
LANES = 16
NUM_WORKERS = NUM_CORES * NUM_SUBCORES

FEAT = 128
ACC_W = 144
K_EDGES = 64
E_PER_W = 10240


def _sc_spmm(x, idx3, n_nodes):
    n_chunks = idx3.shape[1]
    n_pairs = n_chunks // 2
    rows_per_sub = n_nodes // NUM_SUBCORES

    mesh = plsc.VectorSubcoreMesh(core_axis_name="c", subcore_axis_name="s")

    @functools.partial(
        pl.kernel,
        out_type=jax.ShapeDtypeStruct((NUM_CORES, n_nodes, ACC_W), jnp.float32),
        mesh=mesh,
        scratch_types=[
            pltpu.VMEM((2, 3, K_EDGES), jnp.int32),
            pltpu.VMEM((2, K_EDGES), jnp.int32),
            pltpu.VMEM((2, K_EDGES, FEAT), jnp.float32),
            pltpu.VMEM((2, K_EDGES, ACC_W), jnp.float32),
            pltpu.VMEM_SHARED((n_nodes, ACC_W), jnp.float32),
            pltpu.SemaphoreType.DMA,
            pltpu.SemaphoreType.DMA,
            pltpu.SemaphoreType.DMA,
            pltpu.SemaphoreType.DMA,
        ],
        compiler_params=pltpu.CompilerParams(
            use_tc_tiling_on_sc=False, needs_layout_passes=False),
    )
    def spmm(x_hbm, idx3_hbm, out_hbm,
             idx_v, dstbuf, gbuf, sbuf, acc,
             sem_g0, sem_g1, sem_s0, sem_s1):
        c = lax.axis_index("c")
        s = lax.axis_index("s")
        wid = c * NUM_SUBCORES + s
        sem_g = (sem_g0, sem_g1)
        sem_s = (sem_s0, sem_s1)

        zero16 = jnp.zeros((LANES,), jnp.float32)
        lane0 = lax.iota(jnp.int32, LANES) == 0

        def zrow(r, _):
            for j in range(ACC_W // LANES):
                sbuf[0, r, pl.ds(j * LANES, LANES)] = zero16
            return 0
        lax.fori_loop(0, K_EDGES, zrow, 0)
        done = 0
        while done < rows_per_sub:
            step = min(K_EDGES, rows_per_sub - done)
            pltpu.sync_copy(
                sbuf.at[0, pl.ds(0, step)],
                acc.at[pl.ds(s * rows_per_sub + done, step)])
            done += step
        plsc.subcore_barrier()

        def load_idx(g, b):
            pltpu.sync_copy(idx3_hbm.at[wid, g], idx_v.at[b])

        def start_gather(b):
            pltpu.async_copy(x_hbm.at[idx_v.at[b, 0]], gbuf.at[b], sem_g[b])

        def wait_gather(b):
            pltpu.make_async_copy(x_hbm.at[idx_v.at[b, 0]], gbuf.at[b],
                                  sem_g[b]).wait()

        def start_scatter(b):
            pltpu.async_copy(sbuf.at[b], acc.at[dstbuf.at[b]], sem_s[b],
                             add=True)

        def wait_scatter(b):
            pltpu.make_async_copy(sbuf.at[b], acc.at[dstbuf.at[b]],
                                  sem_s[b]).wait()

        def scale(b):
            ew_row = idx_v.at[b, 2]
            nj = FEAT // LANES

            def edge_pair(t, _):
                e0 = 2 * t
                wv0 = plsc.bitcast(plsc.load_gather(
                    ew_row, [jnp.full((LANES,), e0, jnp.int32)]), jnp.float32)
                wv1 = plsc.bitcast(plsc.load_gather(
                    ew_row, [jnp.full((LANES,), e0 + 1, jnp.int32)]),
                    jnp.float32)
                v0 = [gbuf[b, e0, pl.ds(j * LANES, LANES)] for j in range(nj)]
                v1 = [gbuf[b, e0 + 1, pl.ds(j * LANES, LANES)]
                      for j in range(nj)]
                o0 = [v * wv0 for v in v0]
                o1 = [v * wv1 for v in v1]
                for j in range(nj):
                    sbuf[b, e0, pl.ds(j * LANES, LANES)] = o0[j]
                for j in range(nj):
                    sbuf[b, e0 + 1, pl.ds(j * LANES, LANES)] = o1[j]
                sbuf[b, e0, pl.ds(FEAT, LANES)] = jnp.where(lane0, wv0, zero16)
                sbuf[b, e0 + 1, pl.ds(FEAT, LANES)] = jnp.where(
                    lane0, wv1, zero16)
                return 0
            lax.fori_loop(0, K_EDGES // 2, edge_pair, 0)

        def save_dst(b):
            for q in range(K_EDGES // LANES):
                dstbuf[b, pl.ds(q * LANES, LANES)] = (
                    idx_v[b, 1, pl.ds(q * LANES, LANES)])

        load_idx(0, 0)
        load_idx(1, 1)
        start_gather(0)
        start_gather(1)

        def pair(t, _):
            for b in (0, 1):
                g = 2 * t + b
                wait_gather(b)

                @pl.when(t > 0)
                def _():
                    wait_scatter(b)

                scale(b)
                save_dst(b)

                @pl.when(g + 2 < n_chunks)
                def _():
                    load_idx(g + 2, b)
                    start_gather(b)

                start_scatter(b)
            return 0
        lax.fori_loop(0, n_pairs, pair, 0)
        wait_scatter(0)
        wait_scatter(1)
        plsc.subcore_barrier()

        pltpu.sync_copy(
            acc.at[pl.ds(s * rows_per_sub, rows_per_sub)],
            out_hbm.at[c, pl.ds(s * rows_per_sub, rows_per_sub)])

    return spmm(x, idx3)


def _tc_finish(partials, w1, b1, w2, b2):
    n_nodes = partials.shape[1]
    nout = w2.shape[1]

    def body(p_ref, w1_ref, b1_ref, w2_ref, b2_ref, o_ref):
        s = p_ref[0] + p_ref[1]
        w12 = jnp.dot(w1_ref[...], w2_ref[...],
                      preferred_element_type=jnp.float32)
        v = jnp.dot(b1_ref[...], w2_ref[...],
                    preferred_element_type=jnp.float32)
        o_ref[...] = (jnp.dot(s[:, :FEAT], w12,
                              preferred_element_type=jnp.float32)
                      + s[:, FEAT:FEAT + 1] * v + b2_ref[...])

    return pl.pallas_call(
        body,
        out_shape=jax.ShapeDtypeStruct((n_nodes, nout), jnp.float32),
    )(partials, w1, b1.reshape(1, -1), w2, b2.reshape(1, -1))


def kernel(x, edge_index, edge_weight, W1, b1, W2, b2):
    n_nodes = x.shape[0]
    n_edges = edge_index.shape[1]
    e_pad = NUM_WORKERS * E_PER_W - n_edges
    blk = (NUM_WORKERS, E_PER_W // K_EDGES, K_EDGES)
    pad = lambda a: jnp.concatenate([a, jnp.zeros((e_pad,), a.dtype)])
    src = pad(edge_index[1].astype(jnp.int32)).reshape(blk)
    dst = pad(edge_index[0].astype(jnp.int32)).reshape(blk)
    ewb = lax.bitcast_convert_type(
        pad(edge_weight.astype(jnp.float32)), jnp.int32).reshape(blk)
    idx3 = jnp.stack([src, dst, ewb], axis=2)
    partials = _sc_spmm(x, idx3, n_nodes)
    return _tc_finish(partials, W1, b1, W2, b2)


# speedup vs baseline: 6.6885x; 1.4534x over previous
import functools

import jax
import jax.numpy as jnp
from jax import lax
from jax.experimental import pallas as pl
from jax.experimental.pallas import tpu as pltpu
from jax.experimental.pallas import tpu_sc as plsc

NUM_CORES = 2
NUM_SUBCORES = 16
LANES = 16
NUM_WORKERS = NUM_CORES * NUM_SUBCORES

FEAT = 128
ACC_W = 144
K_EDGES = 48
E_PER_W = 10080


def _sc_spmm(x, idx3, n_nodes):
    n_chunks = idx3.shape[1]
    n_pairs = n_chunks // 2
    rows_per_sub = n_nodes // NUM_SUBCORES

    mesh = plsc.VectorSubcoreMesh(core_axis_name="c", subcore_axis_name="s")

    @functools.partial(
        pl.kernel,
        out_type=jax.ShapeDtypeStruct((NUM_CORES, n_nodes, ACC_W), jnp.float32),
        mesh=mesh,
        scratch_types=[
            pltpu.VMEM((3, 3, K_EDGES), jnp.int32),
            pltpu.VMEM((2, K_EDGES), jnp.int32),
            pltpu.VMEM((3, K_EDGES, FEAT), jnp.float32),
            pltpu.VMEM((2, K_EDGES, ACC_W), jnp.float32),
            pltpu.VMEM_SHARED((n_nodes, ACC_W), jnp.float32),
            pltpu.SemaphoreType.DMA,
            pltpu.SemaphoreType.DMA,
            pltpu.SemaphoreType.DMA,
            pltpu.SemaphoreType.DMA,
            pltpu.SemaphoreType.DMA,
            pltpu.SemaphoreType.DMA,
            pltpu.SemaphoreType.DMA,
            pltpu.SemaphoreType.DMA,
        ],
        compiler_params=pltpu.CompilerParams(
            use_tc_tiling_on_sc=False, needs_layout_passes=False),
    )
    def spmm(x_hbm, idx3_hbm, out_hbm,
             idx_v, dstbuf, gbuf, sbuf, acc,
             sem_g0, sem_g1, sem_g2, sem_s0, sem_s1,
             sem_i0, sem_i1, sem_i2):
        c = lax.axis_index("c")
        s = lax.axis_index("s")
        wid = c * NUM_SUBCORES + s
        sem_g = (sem_g0, sem_g1, sem_g2)
        sem_s = (sem_s0, sem_s1)
        sem_i = (sem_i0, sem_i1, sem_i2)

        zero16 = jnp.zeros((LANES,), jnp.float32)
        lane0 = lax.iota(jnp.int32, LANES) == 0

        def zrow(r, _):
            for j in range(ACC_W // LANES):
                sbuf[0, r, pl.ds(j * LANES, LANES)] = zero16
            return 0
        lax.fori_loop(0, K_EDGES, zrow, 0)
        done = 0
        while done < rows_per_sub:
            step = min(K_EDGES, rows_per_sub - done)
            pltpu.sync_copy(
                sbuf.at[0, pl.ds(0, step)],
                acc.at[pl.ds(s * rows_per_sub + done, step)])
            done += step
        plsc.subcore_barrier()

        def start_idx(g, i3):
            pltpu.async_copy(idx3_hbm.at[wid, g], idx_v.at[i3], sem_i[i3])

        def wait_idx(g, i3):
            pltpu.make_async_copy(idx3_hbm.at[wid, g], idx_v.at[i3],
                                  sem_i[i3]).wait()

        def start_gather(i3):
            pltpu.async_copy(x_hbm.at[idx_v.at[i3, 0]], gbuf.at[i3],
                             sem_g[i3])

        def wait_gather(i3):
            pltpu.make_async_copy(x_hbm.at[idx_v.at[i3, 0]], gbuf.at[i3],
                                  sem_g[i3]).wait()

        def start_scatter(b2):
            pltpu.async_copy(sbuf.at[b2], acc.at[dstbuf.at[b2]], sem_s[b2],
                             add=True)

        def wait_scatter(b2):
            pltpu.make_async_copy(sbuf.at[b2], acc.at[dstbuf.at[b2]],
                                  sem_s[b2]).wait()

        def scale(i3, b2):
            ew_row = idx_v.at[i3, 2]
            nj = FEAT // LANES

            def edge_pair(t, _):
                e0 = 2 * t
                wv0 = plsc.bitcast(plsc.load_gather(
                    ew_row, [jnp.full((LANES,), e0, jnp.int32)]), jnp.float32)
                wv1 = plsc.bitcast(plsc.load_gather(
                    ew_row, [jnp.full((LANES,), e0 + 1, jnp.int32)]),
                    jnp.float32)
                v0 = [gbuf[i3, e0, pl.ds(j * LANES, LANES)] for j in range(nj)]
                v1 = [gbuf[i3, e0 + 1, pl.ds(j * LANES, LANES)]
                      for j in range(nj)]
                o0 = [v * wv0 for v in v0]
                o1 = [v * wv1 for v in v1]
                for j in range(nj):
                    sbuf[b2, e0, pl.ds(j * LANES, LANES)] = o0[j]
                for j in range(nj):
                    sbuf[b2, e0 + 1, pl.ds(j * LANES, LANES)] = o1[j]
                sbuf[b2, e0, pl.ds(FEAT, LANES)] = jnp.where(
                    lane0, wv0, zero16)
                sbuf[b2, e0 + 1, pl.ds(FEAT, LANES)] = jnp.where(
                    lane0, wv1, zero16)
                return 0
            lax.fori_loop(0, K_EDGES // 2, edge_pair, 0)

        def save_dst(i3, b2):
            for q in range(K_EDGES // LANES):
                dstbuf[b2, pl.ds(q * LANES, LANES)] = (
                    idx_v[i3, 1, pl.ds(q * LANES, LANES)])

        for g0 in (0, 1, 2):
            start_idx(g0, g0)
        wait_idx(0, 0)
        start_gather(0)
        wait_idx(1, 1)
        start_gather(1)

        def chunk_body(g, u):
            i3, b2 = u % 3, u % 2
            wait_gather(i3)

            @pl.when(g >= 2)
            def _():
                wait_scatter(b2)

            scale(i3, b2)
            save_dst(i3, b2)

            @pl.when(g + 2 < n_chunks)
            def _():
                wait_idx(g + 2, (u + 2) % 3)
                start_gather((u + 2) % 3)

            @pl.when(g + 3 < n_chunks)
            def _():
                start_idx(g + 3, i3)

            start_scatter(b2)

        def six(t, _):
            base = 6 * t
            for u in range(6):
                chunk_body(base + u, u)
            return 0
        lax.fori_loop(0, n_chunks // 6, six, 0)
        wait_scatter(0)
        wait_scatter(1)
        plsc.subcore_barrier()

        pltpu.sync_copy(
            acc.at[pl.ds(s * rows_per_sub, rows_per_sub)],
            out_hbm.at[c, pl.ds(s * rows_per_sub, rows_per_sub)])

    return spmm(x, idx3)


def _tc_finish(partials, w1, b1, w2, b2):
    n_nodes = partials.shape[1]
    nout = w2.shape[1]

    def body(p_ref, w1_ref, b1_ref, w2_ref, b2_ref, o_ref):
        s = p_ref[0] + p_ref[1]
        w12 = jnp.dot(w1_ref[...], w2_ref[...],
                      preferred_element_type=jnp.float32)
        v = jnp.dot(b1_ref[...], w2_ref[...],
                    preferred_element_type=jnp.float32)
        o_ref[...] = (jnp.dot(s[:, :FEAT], w12,
                              preferred_element_type=jnp.float32)
                      + s[:, FEAT:FEAT + 1] * v + b2_ref[...])

    return pl.pallas_call(
        body,
        out_shape=jax.ShapeDtypeStruct((n_nodes, nout), jnp.float32),
    )(partials, w1, b1.reshape(1, -1), w2, b2.reshape(1, -1))


def kernel(x, edge_index, edge_weight, W1, b1, W2, b2):
    n_nodes = x.shape[0]
    n_edges = edge_index.shape[1]
    e_pad = NUM_WORKERS * E_PER_W - n_edges
    blk = (NUM_WORKERS, E_PER_W // K_EDGES, K_EDGES)
    pad = lambda a: jnp.concatenate([a, jnp.zeros((e_pad,), a.dtype)])
    src = pad(edge_index[1].astype(jnp.int32)).reshape(blk)
    dst = pad(edge_index[0].astype(jnp.int32)).reshape(blk)
    ewb = lax.bitcast_convert_type(
        pad(edge_weight.astype(jnp.float32)), jnp.int32).reshape(blk)
    idx3 = jnp.stack([src, dst, ewb], axis=2)
    partials = _sc_spmm(x, idx3, n_nodes)
    return _tc_finish(partials, W1, b1, W2, b2)
